# Initial kernel scaffold; baseline (speedup 1.0000x reference)
import functools

import jax
import jax.numpy as jnp
from jax import lax
from jax.experimental import pallas as pl
from jax.experimental.pallas import tpu as pltpu
from jax.experimental.pallas import tpu_sc as plsc

CH = 128


def _make_gather(n, e, h):
    mesh = plsc.VectorSubcoreMesh(core_axis_name="c", subcore_axis_name="s")
    nw = mesh.num_cores * mesh.num_subcores
    nchunks = e // CH
    full_rounds = nchunks // nw
    rem = nchunks - full_rounds * nw

    @functools.partial(
        pl.kernel,
        out_type=jax.ShapeDtypeStruct((e, h), jnp.float32),
        mesh=mesh,
        scratch_types=[
            pltpu.VMEM((CH,), jnp.int32),
            pltpu.VMEM((CH, h), jnp.float32),
            pltpu.SemaphoreType.DMA,
        ],
    )
    def gather_k(x_hbm, src_hbm, out_hbm, idx_v, rows_v, sem):
        wid = lax.axis_index("s") * mesh.num_cores + lax.axis_index("c")

        def do_chunk(cid):
            off = pl.multiple_of(cid * CH, CH)
            pltpu.sync_copy(src_hbm.at[pl.ds(off, CH)], idx_v)
            pltpu.async_copy(x_hbm.at[idx_v], rows_v, sem).wait()
            pltpu.sync_copy(rows_v, out_hbm.at[pl.ds(off, CH)])

        def body(g, carry):
            do_chunk(g * nw + wid)
            return carry

        lax.fori_loop(0, full_rounds, body, 0)
        if rem:
            @pl.when(wid < rem)
            def _():
                do_chunk(full_rounds * nw + wid)

    return gather_k


def _make_scatter(n, e, h):
    mesh = plsc.VectorSubcoreMesh(core_axis_name="c", subcore_axis_name="s")
    nc, ns = mesh.num_cores, mesh.num_subcores
    nw = nc * ns
    nchunks = e // CH
    full_rounds = nchunks // nw
    rem = nchunks - full_rounds * nw
    rows_per_sub = n // ns

    @functools.partial(
        pl.kernel,
        out_type=jax.ShapeDtypeStruct((nc, n, h), jnp.float32),
        mesh=mesh,
        scratch_types=[
            pltpu.VMEM((CH,), jnp.int32),
            pltpu.VMEM((CH, h), jnp.float32),
            pltpu.VMEM_SHARED((n, h), jnp.float32),
            pltpu.SemaphoreType.DMA,
        ],
    )
    def scatter_k(vals_hbm, dst_hbm, zeros_hbm, out_hbm, idx_v, rows_v,
                  acc_sh, sem):
        cid_ax = lax.axis_index("c")
        sid = lax.axis_index("s")
        wid = sid * nc + cid_ax

        @pl.when(sid == 0)
        def _():
            pltpu.sync_copy(zeros_hbm, acc_sh)

        plsc.subcore_barrier()

        def do_chunk(cid):
            off = pl.multiple_of(cid * CH, CH)
            pltpu.sync_copy(dst_hbm.at[pl.ds(off, CH)], idx_v)
            pltpu.sync_copy(vals_hbm.at[pl.ds(off, CH)], rows_v)
            pltpu.sync_copy(rows_v, acc_sh.at[idx_v], add=True)

        def body(g, carry):
            do_chunk(g * nw + wid)
            return carry

        lax.fori_loop(0, full_rounds, body, 0)
        if rem:
            @pl.when(wid < rem)
            def _():
                do_chunk(full_rounds * nw + wid)

        plsc.subcore_barrier()
        r0 = sid * rows_per_sub
        pltpu.sync_copy(acc_sh.at[pl.ds(r0, rows_per_sub)],
                        out_hbm.at[cid_ax, pl.ds(r0, rows_per_sub)])

    return scatter_k


def _msg_body(xj_ref, ea_ref, w1t_ref, b1_ref, e1_ref, e2_ref, m2_ref,
              b2r_ref, o_ref):
    xj = xj_ref[...]
    hid = jnp.maximum(ea_ref[...] @ w1t_ref[...] + b1_ref[...], 0.0)
    a = xj @ e1_ref[...]
    b = hid @ e2_ref[...]
    o_ref[...] = (a * b) @ m2_ref[...] + xj @ b2r_ref[...]


def _make_msg(e, h, ed, eb):
    grid = e // eb
    full = lambda i: (0, 0)
    return pl.pallas_call(
        _msg_body,
        grid=(grid,),
        in_specs=[
            pl.BlockSpec((eb, h), lambda i: (i, 0)),
            pl.BlockSpec((eb, ed), lambda i: (i, 0)),
            pl.BlockSpec((ed, h), full),
            pl.BlockSpec((1, h), full),
            pl.BlockSpec((h, h * h), full),
            pl.BlockSpec((h, h * h), full),
            pl.BlockSpec((h * h, h), full),
            pl.BlockSpec((h, h), full),
        ],
        out_specs=pl.BlockSpec((eb, h), lambda i: (i, 0)),
        out_shape=jax.ShapeDtypeStruct((e, h), jnp.float32),
    )


def _gru_body(p0_ref, p1_ref, c0_ref, c1_ref, x_ref, root_ref, bias_ref,
              wr_ref, wz_ref, wn_ref, ur_ref, uz_ref, un_ref,
              bir_ref, biz_ref, bin_ref, bhr_ref, bhz_ref, bhn_ref, o_ref):
    x = x_ref[...]
    cnt = c0_ref[...] + c1_ref[...]
    denom = jnp.maximum(cnt, 1.0)
    agg = (p0_ref[...] + p1_ref[...]) / denom
    conv = agg + x @ root_ref[...] + bias_ref[...]
    m = jnp.maximum(conv, 0.0)
    r = jax.nn.sigmoid(m @ wr_ref[...] + bir_ref[...]
                       + x @ ur_ref[...] + bhr_ref[...])
    z = jax.nn.sigmoid(m @ wz_ref[...] + biz_ref[...]
                       + x @ uz_ref[...] + bhz_ref[...])
    nwe = jnp.tanh(m @ wn_ref[...] + bin_ref[...]
                   + r * (x @ un_ref[...] + bhn_ref[...]))
    o_ref[...] = (1.0 - z) * nwe + z * x


def _make_gru(n, h):
    specs = ([pl.BlockSpec((n, h))] * 4
             + [pl.BlockSpec((n, h))]
             + [pl.BlockSpec((h, h)), pl.BlockSpec((1, h))]
             + [pl.BlockSpec((h, h))] * 6
             + [pl.BlockSpec((1, h))] * 6)
    return pl.pallas_call(
        _gru_body,
        in_specs=specs,
        out_specs=pl.BlockSpec((n, h)),
        out_shape=jax.ShapeDtypeStruct((n, h), jnp.float32),
    )


def kernel(out, edge_index, edge_attr, W1, b1, W2, b2, root, bias,
           w_ih, w_hh, b_ih, b_hh):
    n, h = out.shape
    e, ed = edge_attr.shape
    src = edge_index[0]
    dst = edge_index[1]

    w1t = W1.T
    b1r = b1.reshape(1, h)
    w2r3 = W2.reshape(h, h, h)
    m2 = w2r3.transpose(0, 2, 1).reshape(h * h, h)
    b2r = b2.reshape(h, h)
    eye = jnp.eye(h, dtype=jnp.float32)
    e1 = jnp.kron(eye, jnp.ones((1, h), jnp.float32))
    e2 = jnp.kron(jnp.ones((1, h), jnp.float32), eye)
    wr, wz, wn = (w_ih[0:h].T, w_ih[h:2 * h].T, w_ih[2 * h:3 * h].T)
    ur, uz, un = (w_hh[0:h].T, w_hh[h:2 * h].T, w_hh[2 * h:3 * h].T)
    bir, biz, bin_ = (b_ih[0:h].reshape(1, h), b_ih[h:2 * h].reshape(1, h),
                      b_ih[2 * h:3 * h].reshape(1, h))
    bhr, bhz, bhn = (b_hh[0:h].reshape(1, h), b_hh[h:2 * h].reshape(1, h),
                     b_hh[2 * h:3 * h].reshape(1, h))
    biasr = bias.reshape(1, h)
    zeros = jnp.zeros((n, h), jnp.float32)
    ones = jnp.ones((e, h), jnp.float32)

    gather_fn = _make_gather(n, e, h)
    scatter_fn = _make_scatter(n, e, h)
    msg_fn = _make_msg(e, h, ed, eb=1000)
    gru_fn = _make_gru(n, h)

    cntp = scatter_fn(ones, dst, zeros)
    x = out
    for _ in range(3):
        x_j = gather_fn(x, src)
        msg = msg_fn(x_j, edge_attr, w1t, b1r, e1, e2, m2, b2r)
        aggp = scatter_fn(msg, dst, zeros)
        x = gru_fn(aggp[0], aggp[1], cntp[0], cntp[1], x, root, biasr,
                   wr, wz, wn, ur, uz, un, bir, biz, bin_, bhr, bhz, bhn)
    return x


# speedup vs baseline: 1.9698x; 1.9698x over previous
import functools

import jax
import jax.numpy as jnp
from jax import lax
from jax.experimental import pallas as pl
from jax.experimental.pallas import tpu as pltpu
from jax.experimental.pallas import tpu_sc as plsc

CH = 128


def _make_gather(n, e, h):
    mesh = plsc.VectorSubcoreMesh(core_axis_name="c", subcore_axis_name="s")
    nw = mesh.num_cores * mesh.num_subcores
    nchunks = e // CH
    full_rounds = nchunks // nw
    rem = nchunks - full_rounds * nw

    @functools.partial(
        pl.kernel,
        out_type=jax.ShapeDtypeStruct((e, h), jnp.float32),
        mesh=mesh,
        scratch_types=[
            pltpu.VMEM((CH,), jnp.int32),
            pltpu.VMEM((CH, h), jnp.float32),
            pltpu.SemaphoreType.DMA,
        ],
        compiler_params=pltpu.CompilerParams(use_tc_tiling_on_sc=False),
    )
    def gather_k(x_hbm, src_hbm, out_hbm, idx_v, rows_v, sem):
        wid = lax.axis_index("s") * mesh.num_cores + lax.axis_index("c")

        def do_chunk(cid):
            off = pl.multiple_of(cid * CH, CH)
            pltpu.sync_copy(src_hbm.at[pl.ds(off, CH)], idx_v)
            pltpu.async_copy(x_hbm.at[idx_v], rows_v, sem).wait()
            pltpu.sync_copy(rows_v, out_hbm.at[pl.ds(off, CH)])

        def body(g, carry):
            do_chunk(g * nw + wid)
            return carry

        lax.fori_loop(0, full_rounds, body, 0)
        if rem:
            @pl.when(wid < rem)
            def _():
                do_chunk(full_rounds * nw + wid)

    return gather_k


def _make_scatter(n, e, h):
    mesh = plsc.VectorSubcoreMesh(core_axis_name="c", subcore_axis_name="s")
    nc, ns = mesh.num_cores, mesh.num_subcores
    nw = nc * ns
    nchunks = e // CH
    full_rounds = nchunks // nw
    rem = nchunks - full_rounds * nw
    rows_per_sub = n // ns

    @functools.partial(
        pl.kernel,
        out_type=jax.ShapeDtypeStruct((nc, n, h), jnp.float32),
        mesh=mesh,
        scratch_types=[
            pltpu.VMEM((CH,), jnp.int32),
            pltpu.VMEM((CH, h), jnp.float32),
            pltpu.VMEM_SHARED((n, h), jnp.float32),
            pltpu.SemaphoreType.DMA,
        ],
        compiler_params=pltpu.CompilerParams(use_tc_tiling_on_sc=False),
    )
    def scatter_k(vals_hbm, dst_hbm, zeros_hbm, out_hbm, idx_v, rows_v,
                  acc_sh, sem):
        cid_ax = lax.axis_index("c")
        sid = lax.axis_index("s")
        wid = sid * nc + cid_ax

        @pl.when(sid == 0)
        def _():
            pltpu.sync_copy(zeros_hbm, acc_sh)

        plsc.subcore_barrier()

        def do_chunk(cid):
            off = pl.multiple_of(cid * CH, CH)
            pltpu.sync_copy(dst_hbm.at[pl.ds(off, CH)], idx_v)
            pltpu.sync_copy(vals_hbm.at[pl.ds(off, CH)], rows_v)
            pltpu.sync_copy(rows_v, acc_sh.at[idx_v], add=True)

        def body(g, carry):
            do_chunk(g * nw + wid)
            return carry

        lax.fori_loop(0, full_rounds, body, 0)
        if rem:
            @pl.when(wid < rem)
            def _():
                do_chunk(full_rounds * nw + wid)

        plsc.subcore_barrier()
        r0 = sid * rows_per_sub
        pltpu.sync_copy(acc_sh.at[pl.ds(r0, rows_per_sub)],
                        out_hbm.at[cid_ax, pl.ds(r0, rows_per_sub)])

    return scatter_k


def _msg_body(xj_ref, ea_ref, w1t_ref, b1_ref, e1_ref, e2_ref, m2_ref,
              b2r_ref, o_ref):
    xj = xj_ref[...]
    hid = jnp.maximum(ea_ref[...] @ w1t_ref[...] + b1_ref[...], 0.0)
    a = xj @ e1_ref[...]
    b = hid @ e2_ref[...]
    o_ref[...] = (a * b) @ m2_ref[...] + xj @ b2r_ref[...]


def _make_msg(e, h, ed, eb):
    grid = e // eb
    full = lambda i: (0, 0)
    return pl.pallas_call(
        _msg_body,
        grid=(grid,),
        in_specs=[
            pl.BlockSpec((eb, h), lambda i: (i, 0)),
            pl.BlockSpec((eb, ed), lambda i: (i, 0)),
            pl.BlockSpec((ed, h), full),
            pl.BlockSpec((1, h), full),
            pl.BlockSpec((h, h * h), full),
            pl.BlockSpec((h, h * h), full),
            pl.BlockSpec((h * h, h), full),
            pl.BlockSpec((h, h), full),
        ],
        out_specs=pl.BlockSpec((eb, h), lambda i: (i, 0)),
        out_shape=jax.ShapeDtypeStruct((e, h), jnp.float32),
    )


def _gru_body(p0_ref, p1_ref, c0_ref, c1_ref, x_ref, root_ref, bias_ref,
              wr_ref, wz_ref, wn_ref, ur_ref, uz_ref, un_ref,
              bir_ref, biz_ref, bin_ref, bhr_ref, bhz_ref, bhn_ref, o_ref):
    x = x_ref[...]
    cnt = c0_ref[...] + c1_ref[...]
    denom = jnp.maximum(cnt, 1.0)
    agg = (p0_ref[...] + p1_ref[...]) / denom
    conv = agg + x @ root_ref[...] + bias_ref[...]
    m = jnp.maximum(conv, 0.0)
    r = jax.nn.sigmoid(m @ wr_ref[...] + bir_ref[...]
                       + x @ ur_ref[...] + bhr_ref[...])
    z = jax.nn.sigmoid(m @ wz_ref[...] + biz_ref[...]
                       + x @ uz_ref[...] + bhz_ref[...])
    nwe = jnp.tanh(m @ wn_ref[...] + bin_ref[...]
                   + r * (x @ un_ref[...] + bhn_ref[...]))
    o_ref[...] = (1.0 - z) * nwe + z * x


def _make_gru(n, h):
    specs = ([pl.BlockSpec((n, h))] * 4
             + [pl.BlockSpec((n, h))]
             + [pl.BlockSpec((h, h)), pl.BlockSpec((1, h))]
             + [pl.BlockSpec((h, h))] * 6
             + [pl.BlockSpec((1, h))] * 6)
    return pl.pallas_call(
        _gru_body,
        in_specs=specs,
        out_specs=pl.BlockSpec((n, h)),
        out_shape=jax.ShapeDtypeStruct((n, h), jnp.float32),
    )


def kernel(out, edge_index, edge_attr, W1, b1, W2, b2, root, bias,
           w_ih, w_hh, b_ih, b_hh):
    n, h = out.shape
    e, ed = edge_attr.shape
    src = edge_index[0]
    dst = edge_index[1]

    w1t = W1.T
    b1r = b1.reshape(1, h)
    w2r3 = W2.reshape(h, h, h)
    m2 = w2r3.transpose(0, 2, 1).reshape(h * h, h)
    b2r = b2.reshape(h, h)
    eye = jnp.eye(h, dtype=jnp.float32)
    e1 = jnp.kron(eye, jnp.ones((1, h), jnp.float32))
    e2 = jnp.kron(jnp.ones((1, h), jnp.float32), eye)
    wr, wz, wn = (w_ih[0:h].T, w_ih[h:2 * h].T, w_ih[2 * h:3 * h].T)
    ur, uz, un = (w_hh[0:h].T, w_hh[h:2 * h].T, w_hh[2 * h:3 * h].T)
    bir, biz, bin_ = (b_ih[0:h].reshape(1, h), b_ih[h:2 * h].reshape(1, h),
                      b_ih[2 * h:3 * h].reshape(1, h))
    bhr, bhz, bhn = (b_hh[0:h].reshape(1, h), b_hh[h:2 * h].reshape(1, h),
                     b_hh[2 * h:3 * h].reshape(1, h))
    biasr = bias.reshape(1, h)
    zeros = jnp.zeros((n, h), jnp.float32)
    ones = jnp.ones((e, h), jnp.float32)

    gather_fn = _make_gather(n, e, h)
    scatter_fn = _make_scatter(n, e, h)
    msg_fn = _make_msg(e, h, ed, eb=1000)
    gru_fn = _make_gru(n, h)

    cntp = scatter_fn(ones, dst, zeros)
    x = out
    for _ in range(3):
        x_j = gather_fn(x, src)
        msg = msg_fn(x_j, edge_attr, w1t, b1r, e1, e2, m2, b2r)
        aggp = scatter_fn(msg, dst, zeros)
        x = gru_fn(aggp[0], aggp[1], cntp[0], cntp[1], x, root, biasr,
                   wr, wz, wn, ur, uz, un, bir, biz, bin_, bhr, bhz, bhn)
    return x
